# Initial kernel scaffold; baseline (speedup 1.0000x reference)
import jax
import jax.numpy as jnp
from jax.experimental import pallas as pl


def kernel(x, pos, batch, K_points, K_weights):
    raise NotImplementedError("write your pallas kernel here")


# speedup vs baseline: 3.1250x; 3.1250x over previous
import functools

import jax
import jax.numpy as jnp
import numpy as np
from jax import lax
from jax.experimental import pallas as pl
from jax.experimental.pallas import tpu as pltpu
from jax.experimental.pallas import tpu_sc as plsc

N = 10000
K = 16
P = 16
IN_F = 32
OUT_F = 32
RADIUS2 = 1.0
KP_EXTENT = 0.2

RB = 128
NBLK = (N + RB - 1) // RB
NPAD = NBLK * RB
PADPOS = 1.0e6

DG = 48
BTOT = NPAD * K
NWORK = 32
B_PER_W = BTOT // NWORK
CHUNK = 632
NCHUNK_W = B_PER_W // CHUNK


def _knn_body(pos_blk, posT, nbr_ref):
    bx = pos_blk[:, 0:1]
    by = pos_blk[:, 1:2]
    bz = pos_blk[:, 2:3]
    px = posT[0:1, :]
    py = posT[1:2, :]
    pz = posT[2:3, :]
    dx = bx - px
    dy = by - py
    dz = bz - pz
    d2 = dx * dx + dy * dy + dz * dz
    iota_row = lax.broadcasted_iota(jnp.int32, (1, NPAD), 1)
    lane_k = lax.broadcasted_iota(jnp.int32, (1, K), 1)
    nbr = jnp.zeros((RB, K), jnp.int32)
    for k in range(K):
        m = jnp.min(d2, axis=1, keepdims=True)
        idx = jnp.min(jnp.where(d2 == m, iota_row, NPAD), axis=1, keepdims=True)
        nbr = jnp.where(lane_k == k, idx, nbr)
        d2 = jnp.where(iota_row == idx, jnp.float32(np.inf), d2)
    nbr_ref[...] = nbr


def _knn(pos_pad, posT8):
    return pl.pallas_call(
        _knn_body,
        grid=(NBLK,),
        in_specs=[
            pl.BlockSpec((RB, 3), lambda i: (i, 0)),
            pl.BlockSpec((8, NPAD), lambda i: (0, 0)),
        ],
        out_specs=pl.BlockSpec((RB, K), lambda i: (i, 0)),
        out_shape=jax.ShapeDtypeStruct((NPAD, K), jnp.int32),
    )(pos_pad, posT8)


def _gather(table, idx_flat):
    mesh = plsc.VectorSubcoreMesh(core_axis_name="c", subcore_axis_name="s")

    @functools.partial(
        pl.kernel,
        out_type=jax.ShapeDtypeStruct((BTOT, DG), jnp.float32),
        mesh=mesh,
        scratch_types=[
            pltpu.VMEM((CHUNK,), jnp.int32),
            pltpu.VMEM((CHUNK, DG), jnp.float32),
            pltpu.SemaphoreType.DMA,
        ],
        compiler_params=pltpu.CompilerParams(use_tc_tiling_on_sc=False),
    )
    def gk(table_hbm, idx_hbm, out_hbm, idx_v, rows_v, sem):
        wid = lax.axis_index("s") * 2 + lax.axis_index("c")
        base = wid * B_PER_W

        def body(c, carry):
            off = base + c * CHUNK
            pltpu.sync_copy(idx_hbm.at[pl.ds(off, CHUNK)], idx_v)
            pltpu.async_copy(table_hbm.at[idx_v], rows_v, sem).wait()
            pltpu.sync_copy(rows_v, out_hbm.at[pl.ds(off, CHUNK)])
            return carry

        lax.fori_loop(0, NCHUNK_W, body, 0)

    return gk(table, idx_flat)


def _conv_body(xj, pos_rep, kt, w_cat, b_sum, seg, out_ref):
    feats = xj[:, 0:IN_F]
    dx = xj[:, 32:33] - pos_rep[:, 0:1]
    dy = xj[:, 33:34] - pos_rep[:, 1:2]
    dz = xj[:, 34:35] - pos_rep[:, 2:3]
    maskc = ((dx * dx + dy * dy + dz * dz) <= RADIUS2).astype(jnp.float32)
    ddx = dx - kt[0:1, :]
    ddy = dy - kt[1:2, :]
    ddz = dz - kt[2:3, :]
    nrm = jnp.sqrt(ddx * ddx + ddy * ddy + ddz * ddz)
    den = jnp.maximum(nrm, 1e-12)
    s = ddx / den + ddy / den + ddz / den
    w = 1.0 - jnp.sqrt(jnp.maximum(s, 0.0)) / KP_EXTENT
    w = jnp.where(w < 0.0, 0.0, w)
    mrow = jnp.min(s, axis=1, keepdims=True)
    i16 = lax.broadcasted_iota(jnp.int32, (1, P), 1)
    cidx = jnp.min(jnp.where(s == mrow, i16, P), axis=1, keepdims=True)
    w_sel = jnp.sum(jnp.where(i16 == cidx, w, 0.0), axis=1, keepdims=True)
    wx = feats * (w_sel * maskc)
    y = jnp.dot(wx, w_cat[...], precision=lax.Precision.HIGHEST)
    lane_p = lax.broadcasted_iota(jnp.int32, (1, P * OUT_F), 1) // OUT_F
    selv = jnp.where(lane_p == cidx, y, 0.0)
    msg = jnp.dot(selv, b_sum[...], precision=lax.Precision.HIGHEST)
    segm = seg[...]
    acc = jnp.dot(segm, msg, precision=lax.Precision.HIGHEST)
    cnt = jnp.dot(segm, maskc, precision=lax.Precision.HIGHEST)
    out_ref[...] = acc / jnp.maximum(cnt, 1.0)


def _conv(xj, pos_rep, kt8, w_cat, b_sum, seg):
    return pl.pallas_call(
        _conv_body,
        grid=(NBLK,),
        in_specs=[
            pl.BlockSpec((RB * K, DG), lambda i: (i, 0)),
            pl.BlockSpec((RB * K, 3), lambda i: (i, 0)),
            pl.BlockSpec((8, P), lambda i: (0, 0)),
            pl.BlockSpec((IN_F, P * OUT_F), lambda i: (0, 0)),
            pl.BlockSpec((P * OUT_F, OUT_F), lambda i: (0, 0)),
            pl.BlockSpec((RB, RB * K), lambda i: (0, 0)),
        ],
        out_specs=pl.BlockSpec((RB, OUT_F), lambda i: (i, 0)),
        out_shape=jax.ShapeDtypeStruct((NPAD, OUT_F), jnp.float32),
    )(xj, pos_rep, kt8, w_cat, b_sum, seg)


def kernel(x, pos, batch, K_points, K_weights):
    del batch
    pos_pad = jnp.full((NPAD, 3), PADPOS, jnp.float32).at[:N].set(pos)
    posT8 = jnp.zeros((8, NPAD), jnp.float32).at[:3].set(pos_pad.T)

    nbr = _knn(pos_pad, posT8)
    idx_flat = jnp.minimum(nbr.reshape(-1), N - 1)

    table = jnp.zeros((N, DG), jnp.float32)
    table = table.at[:, :IN_F].set(x).at[:, 32:35].set(pos)
    xj = _gather(table, idx_flat)

    pos_rep = jnp.repeat(pos_pad, K, axis=0)
    kt8 = jnp.zeros((8, P), jnp.float32).at[:3].set(K_points.T)
    w_cat = jnp.transpose(K_weights, (2, 0, 1)).reshape(IN_F, P * OUT_F)
    b_sum = (jnp.arange(P * OUT_F)[:, None] % OUT_F
             == jnp.arange(OUT_F)[None, :]).astype(jnp.float32)
    seg = (jnp.arange(RB)[:, None]
           == jnp.arange(RB * K)[None, :] // K).astype(jnp.float32)

    out = _conv(xj, pos_rep, kt8, w_cat, b_sum, seg)
    return out[:N]


# speedup vs baseline: 5.1557x; 1.6498x over previous
import functools

import jax
import jax.numpy as jnp
import numpy as np
from jax import lax
from jax.experimental import pallas as pl
from jax.experimental.pallas import tpu as pltpu
from jax.experimental.pallas import tpu_sc as plsc

N = 10000
K = 16
P = 16
IN_F = 32
OUT_F = 32
RADIUS2 = 1.0
KP_EXTENT = 0.2

RB = 128
NBLK = 80
NPAD = NBLK * RB
PADPOS = 1.0e6

RBC = 256
NBLKC = NPAD // RBC

DG = 48
BTOT = NPAD * K
NWORK = 32
B_PER_W = BTOT // NWORK
CHUNK = 640
NCHUNK_W = B_PER_W // CHUNK


def _knn_body(pos_blk, posT, nbr_ref):
    bx = pos_blk[:, 0:1]
    by = pos_blk[:, 1:2]
    bz = pos_blk[:, 2:3]
    px = posT[0:1, :]
    py = posT[1:2, :]
    pz = posT[2:3, :]
    dx = bx - px
    dy = by - py
    dz = bz - pz
    d2 = dx * dx + dy * dy + dz * dz
    iota_row = lax.broadcasted_iota(jnp.int32, (1, NPAD), 1)
    lane_k = lax.broadcasted_iota(jnp.int32, (1, K), 1)
    nbr = jnp.zeros((RB, K), jnp.int32)
    for k in range(K):
        m = jnp.min(d2, axis=1, keepdims=True)
        idx = jnp.min(jnp.where(d2 == m, iota_row, NPAD), axis=1, keepdims=True)
        nbr = jnp.where(lane_k == k, idx, nbr)
        d2 = jnp.where(iota_row == idx, jnp.float32(np.inf), d2)
    nbr_ref[...] = nbr


def _knn(pos_pad, posT8):
    return pl.pallas_call(
        _knn_body,
        grid=(NBLK,),
        in_specs=[
            pl.BlockSpec((RB, 3), lambda i: (i, 0)),
            pl.BlockSpec((8, NPAD), lambda i: (0, 0)),
        ],
        out_specs=pl.BlockSpec((RB, K), lambda i: (i, 0)),
        out_shape=jax.ShapeDtypeStruct((NPAD, K), jnp.int32),
    )(pos_pad, posT8)


def _gather(table, idx_flat):
    mesh = plsc.VectorSubcoreMesh(core_axis_name="c", subcore_axis_name="s")

    @functools.partial(
        pl.kernel,
        out_type=jax.ShapeDtypeStruct((BTOT, DG), jnp.float32),
        mesh=mesh,
        scratch_types=[
            pltpu.VMEM((CHUNK,), jnp.int32),
            pltpu.VMEM((CHUNK, DG), jnp.float32),
            pltpu.SemaphoreType.DMA,
        ],
        compiler_params=pltpu.CompilerParams(use_tc_tiling_on_sc=False),
    )
    def gk(table_hbm, idx_hbm, out_hbm, idx_v, rows_v, sem):
        wid = lax.axis_index("s") * 2 + lax.axis_index("c")
        base = wid * B_PER_W

        def body(c, carry):
            off = base + c * CHUNK
            pltpu.sync_copy(idx_hbm.at[pl.ds(off, CHUNK)], idx_v)
            pltpu.async_copy(table_hbm.at[idx_v], rows_v, sem).wait()
            pltpu.sync_copy(rows_v, out_hbm.at[pl.ds(off, CHUNK)])
            return carry

        lax.fori_loop(0, NCHUNK_W, body, 0)

    return gk(table, idx_flat)


def _conv_body(xj, pos_rep, kt, w_cat2, out_ref):
    feats = xj[:, 0:IN_F]
    dx = xj[:, 32:33] - pos_rep[:, 0:1]
    dy = xj[:, 33:34] - pos_rep[:, 1:2]
    dz = xj[:, 34:35] - pos_rep[:, 2:3]
    maskc = ((dx * dx + dy * dy + dz * dz) <= RADIUS2).astype(jnp.float32)
    ddx = dx - kt[0:1, :]
    ddy = dy - kt[1:2, :]
    ddz = dz - kt[2:3, :]
    nrm = jnp.sqrt(ddx * ddx + ddy * ddy + ddz * ddz)
    den = jnp.maximum(nrm, 1e-12)
    s = ddx / den + ddy / den + ddz / den
    w = 1.0 - jnp.sqrt(jnp.maximum(s, 0.0)) / KP_EXTENT
    w = jnp.where(w < 0.0, 0.0, w)
    mrow = jnp.min(s, axis=1, keepdims=True)
    i16 = lax.broadcasted_iota(jnp.int32, (1, P), 1)
    cidx = jnp.min(jnp.where(s == mrow, i16, P), axis=1, keepdims=True)
    w_sel = jnp.sum(jnp.where(i16 == cidx, w, 0.0), axis=1, keepdims=True)
    wx = feats * (w_sel * maskc)
    wxt = jnp.concatenate([wx] * P, axis=1)
    lane_p = lax.broadcasted_iota(jnp.int32, (1, P * OUT_F), 1) // OUT_F
    xhat = jnp.where(lane_p == cidx, wxt, 0.0)
    agg = jnp.sum(xhat.reshape(RBC, K, P * IN_F), axis=1)
    acc = jnp.dot(agg, w_cat2[...])
    cnt = jnp.sum(maskc.reshape(RBC, K, 1), axis=1)
    out_ref[...] = acc / jnp.maximum(cnt, 1.0)


def _conv(xj, pos_rep, kt8, w_cat2):
    return pl.pallas_call(
        _conv_body,
        grid=(NBLKC,),
        in_specs=[
            pl.BlockSpec((RBC * K, DG), lambda i: (i, 0)),
            pl.BlockSpec((RBC * K, 3), lambda i: (i, 0)),
            pl.BlockSpec((8, P), lambda i: (0, 0)),
            pl.BlockSpec((P * IN_F, OUT_F), lambda i: (0, 0)),
        ],
        out_specs=pl.BlockSpec((RBC, OUT_F), lambda i: (i, 0)),
        out_shape=jax.ShapeDtypeStruct((NPAD, OUT_F), jnp.float32),
    )(xj, pos_rep, kt8, w_cat2)


def kernel(x, pos, batch, K_points, K_weights):
    del batch
    pos_pad = jnp.full((NPAD, 3), PADPOS, jnp.float32).at[:N].set(pos)
    posT8 = jnp.zeros((8, NPAD), jnp.float32).at[:3].set(pos_pad.T)

    nbr = _knn(pos_pad, posT8)
    idx_flat = jnp.minimum(nbr.reshape(-1), N - 1)

    table = jnp.zeros((N, DG), jnp.float32)
    table = table.at[:, :IN_F].set(x).at[:, 32:35].set(pos)
    xj = _gather(table, idx_flat)

    pos_rep = jnp.repeat(pos_pad, K, axis=0)
    kt8 = jnp.zeros((8, P), jnp.float32).at[:3].set(K_points.T)
    w_cat2 = jnp.transpose(K_weights, (0, 2, 1)).reshape(P * IN_F, OUT_F)

    out = _conv(xj, pos_rep, kt8, w_cat2)
    return out[:N]
